# baseline (speedup 1.0000x reference)
import functools

import jax
import jax.numpy as jnp
from jax import lax
from jax.experimental import pallas as pl
from jax.experimental.pallas import tpu as pltpu
from jax.experimental.pallas import tpu_sc as plsc

NN = 10000
NP = 10240
E = 640000
NC, NS = 2, 16
NW = NC * NS
K = 128
NBC = 160
EWP = NBC * K
EP = NW * EWP
SL = NP // NS
ZB = SL // K
WF = 128
F32 = jnp.float32


def _make_edge_agg(width):
    if width == 1:
        out_t = jax.ShapeDtypeStruct((NC, NP), F32)
        rows_t = pltpu.VMEM((2, K), F32)
        acc_t = pltpu.VMEM_SHARED((NP,), F32)
    else:
        out_t = jax.ShapeDtypeStruct((NC, NP, width), F32)
        rows_t = pltpu.VMEM((2, K, width), F32)
        acc_t = pltpu.VMEM_SHARED((NP, width), F32)

    mesh = plsc.VectorSubcoreMesh(
        core_axis_name="c", subcore_axis_name="s", num_cores=NC, num_subcores=NS
    )

    @functools.partial(
        pl.kernel,
        out_type=out_t,
        mesh=mesh,
        scratch_types=[
            pltpu.VMEM((2, K), jnp.int32),
            pltpu.VMEM((4, K), jnp.int32),
            rows_t,
            acc_t,
            pltpu.SemaphoreType.DMA,
            pltpu.SemaphoreType.DMA,
            pltpu.SemaphoreType.DMA,
            pltpu.SemaphoreType.DMA,
            pltpu.SemaphoreType.DMA,
            pltpu.SemaphoreType.DMA,
            pltpu.SemaphoreType.DMA,
            pltpu.SemaphoreType.DMA,
            pltpu.SemaphoreType.DMA,
            pltpu.SemaphoreType.DMA,
        ],
    )
    def agg(h_hbm, src_hbm, dst_hbm, out_hbm, sb, db, rows, acc,
            is0, is1, ds0, ds1, ds2, ds3, gs0, gs1, ss0, ss1):
        c = lax.axis_index("c")
        s = lax.axis_index("s")
        w = c * NS + s
        isem = (is0, is1)
        dsem = (ds0, ds1, ds2, ds3)
        gsem = (gs0, gs1)
        ssem = (ss0, ss1)
        r = (rows.at[0], rows.at[1])

        def st_src(b, p):
            pltpu.async_copy(src_hbm.at[w, b], sb.at[p], isem[p])

        def st_dst(b, q):
            pltpu.async_copy(dst_hbm.at[w, b], db.at[q], dsem[q])

        def w_src(b, p):
            pltpu.make_async_copy(src_hbm.at[w, b], sb.at[p], isem[p]).wait()

        def w_dst(b, q):
            pltpu.make_async_copy(dst_hbm.at[w, b], db.at[q], dsem[q]).wait()

        def fire_g(p):
            pltpu.async_copy(h_hbm.at[sb.at[p]], r[p], gsem[p])

        def w_g(p):
            pltpu.make_async_copy(h_hbm.at[sb.at[p]], r[p], gsem[p]).wait()

        def fire_s(p, q):
            pltpu.async_copy(r[p], acc.at[db.at[q]], ssem[p], add=True)

        def w_s(p, q):
            pltpu.make_async_copy(r[p], acc.at[db.at[q]], ssem[p]).wait()

        def batch_iter(b, pr, pd, ss_wait=True, fire_next=True,
                       src_stage=True, dst_stage=True):
            pn = 1 - pr
            if ss_wait:
                w_s(pn, (pd + 3) % 4)
            if fire_next:
                w_src(b + 1, pn)
                fire_g(pn)
            if dst_stage:
                st_dst(b + 3, (pd + 3) % 4)
            w_g(pr)
            if src_stage:
                st_src(b + 2, pr)
            w_dst(b, pd)
            fire_s(pr, pd)

        st_src(0, 0)
        st_src(1, 1)
        st_dst(0, 0)
        st_dst(1, 1)
        st_dst(2, 2)
        if width == 1:
            r[0][...] = jnp.zeros((K,), F32)
            for z in range(ZB):
                pltpu.sync_copy(r[0], acc.at[pl.ds(s * SL + z * K, K)])
        else:
            r[0][...] = jnp.zeros((K, width), F32)
            for z in range(ZB):
                pltpu.sync_copy(r[0], acc.at[pl.ds(s * SL + z * K, K), :])
        w_src(0, 0)
        fire_g(0)
        plsc.subcore_barrier()

        batch_iter(0, 0, 0, ss_wait=False)

        def step(j, carry):
            b = 4 * j + 1
            batch_iter(b, 1, 1)
            batch_iter(b + 1, 0, 2)
            batch_iter(b + 2, 1, 3)
            batch_iter(b + 3, 0, 0)
            return carry

        lax.fori_loop(0, (NBC - 4) // 4, step, 0)

        batch_iter(NBC - 3, 1, 1, dst_stage=False)
        batch_iter(NBC - 2, 0, 2, dst_stage=False, src_stage=False)
        batch_iter(NBC - 1, 1, 3, dst_stage=False, src_stage=False,
                   fire_next=False)
        w_s(1, 3)

        plsc.subcore_barrier()
        if width == 1:
            pltpu.sync_copy(acc.at[pl.ds(s * SL, SL)], out_hbm.at[c, pl.ds(s * SL, SL)])
        else:
            pltpu.sync_copy(
                acc.at[pl.ds(s * SL, SL), :], out_hbm.at[c, pl.ds(s * SL, SL), :]
            )

    return agg


_agg1 = _make_edge_agg(1)
_aggF = _make_edge_agg(WF)


_R = 2048


_NR = NP // 128


def _prep_body(u_ref, y_ref, dinv_ref, hs_ref):
    indeg = u_ref[0] + u_ref[1]
    dinv = lax.rsqrt(indeg + 1.0)
    dinv_ref[...] = dinv
    hs_ref[...] = dinv * y_ref[...]


def _prep(u, y):
    return pl.pallas_call(
        _prep_body,
        out_shape=(
            jax.ShapeDtypeStruct((_NR, 128), F32),
            jax.ShapeDtypeStruct((_NR, 128), F32),
        ),
    )(u, y)


def _l0_body(u_ref, hs_ref, dinv_ref, w_ref, b_ref, o_ref):
    dinv = dinv_ref[...]
    agg = dinv * (u_ref[0] + u_ref[1] + hs_ref[...])
    h = jnp.tanh(agg * w_ref[...] + b_ref[...])
    o_ref[...] = dinv * h


def _l0(u, hs, dinv, w0, b0):
    g = NP // _R
    return pl.pallas_call(
        _l0_body,
        grid=(g,),
        in_specs=[
            pl.BlockSpec((2, _R, 1), lambda i: (0, i, 0)),
            pl.BlockSpec((_R, 1), lambda i: (i, 0)),
            pl.BlockSpec((_R, 1), lambda i: (i, 0)),
            pl.BlockSpec((1, WF), lambda i: (0, 0)),
            pl.BlockSpec((1, WF), lambda i: (0, 0)),
        ],
        out_specs=pl.BlockSpec((_R, WF), lambda i: (i, 0)),
        out_shape=jax.ShapeDtypeStruct((NP, WF), F32),
    )(u, hs, dinv, w0, b0)


def _lmid_body(u_ref, hs_ref, dinv_ref, w_ref, b_ref, o_ref):
    dinv = dinv_ref[...]
    agg = dinv * (u_ref[0] + u_ref[1] + hs_ref[...])
    h = jnp.tanh(
        jnp.dot(agg, w_ref[...], preferred_element_type=F32) + b_ref[...]
    )
    o_ref[...] = dinv * h


def _lmid(u, hs, dinv, w, b):
    g = NP // _R
    return pl.pallas_call(
        _lmid_body,
        grid=(g,),
        in_specs=[
            pl.BlockSpec((2, _R, WF), lambda i: (0, i, 0)),
            pl.BlockSpec((_R, WF), lambda i: (i, 0)),
            pl.BlockSpec((_R, 1), lambda i: (i, 0)),
            pl.BlockSpec((WF, WF), lambda i: (0, 0)),
            pl.BlockSpec((1, WF), lambda i: (0, 0)),
        ],
        out_specs=pl.BlockSpec((_R, WF), lambda i: (i, 0)),
        out_shape=jax.ShapeDtypeStruct((NP, WF), F32),
    )(u, hs, dinv, w, b)


def _l3z_body(u_ref, hs_ref, dinv_ref, w3_ref, b3_ref, w4_ref, o_ref):
    dinv = dinv_ref[...]
    agg = dinv * (u_ref[0] + u_ref[1] + hs_ref[...])
    h4 = jnp.tanh(
        jnp.dot(agg, w3_ref[...], preferred_element_type=F32) + b3_ref[...]
    )
    z = jnp.dot(h4, w4_ref[...], preferred_element_type=F32)
    o_ref[...] = dinv * z


def _l3z(u, hs, dinv, w3, b3, w4):
    g = NP // _R
    return pl.pallas_call(
        _l3z_body,
        grid=(g,),
        in_specs=[
            pl.BlockSpec((2, _R, WF), lambda i: (0, i, 0)),
            pl.BlockSpec((_R, WF), lambda i: (i, 0)),
            pl.BlockSpec((_R, 1), lambda i: (i, 0)),
            pl.BlockSpec((WF, WF), lambda i: (0, 0)),
            pl.BlockSpec((1, WF), lambda i: (0, 0)),
            pl.BlockSpec((WF, 1), lambda i: (0, 0)),
        ],
        out_specs=pl.BlockSpec((_R, 1), lambda i: (i, 0)),
        out_shape=jax.ShapeDtypeStruct((NP, 1), F32),
    )(u, hs, dinv, w3, b3, w4)


def _fin_body(u_ref, zs_ref, dinv_ref, b4_ref, y_ref, ka_ref, kb_ref, kc_ref,
              coef_ref, k_ref, ya_ref, hsa_ref):
    dinv = dinv_ref[...]
    k = dinv * (u_ref[0] + u_ref[1] + zs_ref[...]) + b4_ref[0, 0]
    y_arg = (
        y_ref[...]
        + coef_ref[0, 0] * ka_ref[...]
        + coef_ref[0, 1] * kb_ref[...]
        + coef_ref[0, 2] * kc_ref[...]
        + coef_ref[0, 3] * k
    )
    k_ref[...] = k
    ya_ref[...] = y_arg
    hsa_ref[...] = dinv * y_arg


def _fin(u, zs, dinv, b4, y, ka, kb, kc, coef):
    return pl.pallas_call(
        _fin_body,
        out_shape=(
            jax.ShapeDtypeStruct((_NR, 128), F32),
            jax.ShapeDtypeStruct((_NR, 128), F32),
            jax.ShapeDtypeStruct((_NR, 128), F32),
        ),
    )(u, zs, dinv, b4, y, ka, kb, kc, coef)


def _padw(w, rows, cols):
    return jnp.pad(w, ((0, rows - w.shape[0]), (0, cols - w.shape[1])))


def kernel(x, edge_index, W0, b0, W1, b1, W2, b2, W3, b3, W4, b4):
    bsz, nn, _ = x.shape
    n = bsz * nn
    pad = NP - n
    y0 = x[:, :, -1].reshape(n)
    y2 = jnp.pad(y0, (0, pad)).reshape(_NR, 128)
    epad = jnp.full((EP - E,), NP - 1, jnp.int32)
    src = jnp.concatenate([edge_index[0].astype(jnp.int32), epad]).reshape(NW, NBC, K)
    dst = jnp.concatenate([edge_index[1].astype(jnp.int32), epad]).reshape(NW, NBC, K)

    w0p = _padw(W0, 1, WF)
    w1p = _padw(W1, WF, WF)
    w2p = _padw(W2, WF, WF)
    w3p = _padw(W3, WF, WF)
    w4p = _padw(W4.reshape(64, 1), WF, 1)
    b0p, b1p, b2p, b3p = (
        jnp.pad(b.reshape(1, 64), ((0, 0), (0, WF - 64))) for b in (b0, b1, b2, b3)
    )
    b4r = b4.reshape(1, 1)

    deg = _agg1(jnp.ones((NP,), F32), src, dst)
    dinv2, hs2 = _prep(deg.reshape(NC, _NR, 128), y2)
    dinvcol = dinv2.reshape(NP, 1)

    dt = 1.25
    coefs = [
        (0.0, 0.0, 0.0, dt / 3.0),
        (-dt / 3.0, 0.0, 0.0, dt),
        (dt, -dt, 0.0, dt),
        (dt / 8.0, 3.0 * dt / 8.0, 3.0 * dt / 8.0, dt / 8.0),
    ]
    coefs = [jnp.asarray(c, F32).reshape(1, 4) for c in coefs]

    preds = [y0]
    for _ in range(4):
        ks = []
        for e in range(4):
            u = _agg1(hs2.reshape(NP), src, dst).reshape(NC, NP, 1)
            h1 = _l0(u, hs2.reshape(NP, 1), dinvcol, w0p, b0p)
            u = _aggF(h1, src, dst)
            h2 = _lmid(u, h1, dinvcol, w1p, b1p)
            u = _aggF(h2, src, dst)
            h3 = _lmid(u, h2, dinvcol, w2p, b2p)
            u = _aggF(h3, src, dst)
            zs = _l3z(u, h3, dinvcol, w3p, b3p, w4p)
            u = _agg1(zs.reshape(NP), src, dst).reshape(NC, _NR, 128)
            pads = [y2, y2, y2]
            ka, kb, kc = (ks + pads)[:3]
            knew, ya2, hs2 = _fin(
                u, zs.reshape(_NR, 128), dinv2, b4r, y2, ka, kb, kc, coefs[e]
            )
            ks.append(knew)
        y2 = ya2
        preds.append(y2.reshape(NP)[:n])
    out = jnp.stack([p.reshape(nn) for p in preds], axis=-1)
    return out[None].astype(x.dtype)


# speedup vs baseline: 1.0002x; 1.0002x over previous
import functools

import jax
import jax.numpy as jnp
from jax import lax
from jax.experimental import pallas as pl
from jax.experimental.pallas import tpu as pltpu
from jax.experimental.pallas import tpu_sc as plsc

NN = 10000
NP = 10240
E = 640000
NC, NS = 2, 16
NW = NC * NS
K = 128
NBC = 160
EWP = NBC * K
EP = NW * EWP
SL = NP // NS
ZB = SL // K
WF = 128
F32 = jnp.float32


def _make_edge_agg(width):
    if width == 1:
        out_t = jax.ShapeDtypeStruct((NC, NP), F32)
        rows_t = pltpu.VMEM((2, K), F32)
        acc_t = pltpu.VMEM_SHARED((NP,), F32)
    else:
        out_t = jax.ShapeDtypeStruct((NC, NP, width), F32)
        rows_t = pltpu.VMEM((2, K, width), F32)
        acc_t = pltpu.VMEM_SHARED((NP, width), F32)

    mesh = plsc.VectorSubcoreMesh(
        core_axis_name="c", subcore_axis_name="s", num_cores=NC, num_subcores=NS
    )

    @functools.partial(
        pl.kernel,
        out_type=out_t,
        mesh=mesh,
        scratch_types=[
            pltpu.VMEM((2, K), jnp.int32),
            pltpu.VMEM((4, K), jnp.int32),
            rows_t,
            acc_t,
            pltpu.SemaphoreType.DMA,
            pltpu.SemaphoreType.DMA,
            pltpu.SemaphoreType.DMA,
            pltpu.SemaphoreType.DMA,
            pltpu.SemaphoreType.DMA,
            pltpu.SemaphoreType.DMA,
            pltpu.SemaphoreType.DMA,
            pltpu.SemaphoreType.DMA,
            pltpu.SemaphoreType.DMA,
            pltpu.SemaphoreType.DMA,
        ],
    )
    def agg(h_hbm, src_hbm, dst_hbm, out_hbm, sb, db, rows, acc,
            is0, is1, ds0, ds1, ds2, ds3, gs0, gs1, ss0, ss1):
        c = lax.axis_index("c")
        s = lax.axis_index("s")
        w = c * NS + s
        isem = (is0, is1)
        dsem = (ds0, ds1, ds2, ds3)
        gsem = (gs0, gs1)
        ssem = (ss0, ss1)
        r = (rows.at[0], rows.at[1])

        def st_src(b, p):
            pltpu.async_copy(src_hbm.at[w, b], sb.at[p], isem[p])

        def st_dst(b, q):
            pltpu.async_copy(dst_hbm.at[w, b], db.at[q], dsem[q])

        def w_src(b, p):
            pltpu.make_async_copy(src_hbm.at[w, b], sb.at[p], isem[p]).wait()

        def w_dst(b, q):
            pltpu.make_async_copy(dst_hbm.at[w, b], db.at[q], dsem[q]).wait()

        def fire_g(p):
            pltpu.async_copy(h_hbm.at[sb.at[p]], r[p], gsem[p])

        def w_g(p):
            pltpu.make_async_copy(h_hbm.at[sb.at[p]], r[p], gsem[p]).wait()

        def scat(p, q):
            pltpu.sync_copy(r[p], acc.at[db.at[q]], add=True)

        def pair(i, q0, q1):
            w_src(i + 1, 1)
            fire_g(1)
            w_g(0)
            st_src(i + 2, 0)
            w_dst(i, q0)
            scat(0, q0)
            st_dst(i + 4, q0)
            w_src(i + 2, 0)
            fire_g(0)
            w_g(1)
            st_src(i + 3, 1)
            w_dst(i + 1, q1)
            scat(1, q1)
            st_dst(i + 5, q1)

        st_src(0, 0)
        st_src(1, 1)
        st_dst(0, 0)
        st_dst(1, 1)
        st_dst(2, 2)
        st_dst(3, 3)
        if width == 1:
            r[0][...] = jnp.zeros((K,), F32)
            for z in range(ZB):
                pltpu.sync_copy(r[0], acc.at[pl.ds(s * SL + z * K, K)])
        else:
            r[0][...] = jnp.zeros((K, width), F32)
            for z in range(ZB):
                pltpu.sync_copy(r[0], acc.at[pl.ds(s * SL + z * K, K), :])
        w_src(0, 0)
        fire_g(0)
        plsc.subcore_barrier()

        def step(j, carry):
            i = 4 * j
            pair(i, 0, 1)
            pair(i + 2, 2, 3)
            return carry

        lax.fori_loop(0, (NBC - 4) // 4, step, 0)

        i = NBC - 4
        w_src(i + 1, 1)
        fire_g(1)
        w_g(0)
        st_src(i + 2, 0)
        w_dst(i, 0)
        scat(0, 0)
        w_src(i + 2, 0)
        fire_g(0)
        w_g(1)
        st_src(i + 3, 1)
        w_dst(i + 1, 1)
        scat(1, 1)
        w_src(i + 3, 1)
        fire_g(1)
        w_g(0)
        w_dst(i + 2, 2)
        scat(0, 2)
        w_g(1)
        w_dst(i + 3, 3)
        scat(1, 3)

        plsc.subcore_barrier()
        if width == 1:
            pltpu.sync_copy(acc.at[pl.ds(s * SL, SL)], out_hbm.at[c, pl.ds(s * SL, SL)])
        else:
            pltpu.sync_copy(
                acc.at[pl.ds(s * SL, SL), :], out_hbm.at[c, pl.ds(s * SL, SL), :]
            )

    return agg


_agg1 = _make_edge_agg(1)
_aggF = _make_edge_agg(WF)


_R = 2048


_NR = NP // 128


def _prep_body(u_ref, y_ref, dinv_ref, hs_ref):
    indeg = u_ref[0] + u_ref[1]
    dinv = lax.rsqrt(indeg + 1.0)
    dinv_ref[...] = dinv
    hs_ref[...] = dinv * y_ref[...]


def _prep(u, y):
    return pl.pallas_call(
        _prep_body,
        out_shape=(
            jax.ShapeDtypeStruct((_NR, 128), F32),
            jax.ShapeDtypeStruct((_NR, 128), F32),
        ),
    )(u, y)


def _l0_body(u_ref, hs_ref, dinv_ref, w_ref, b_ref, o_ref):
    dinv = dinv_ref[...]
    agg = dinv * (u_ref[0] + u_ref[1] + hs_ref[...])
    h = jnp.tanh(agg * w_ref[...] + b_ref[...])
    o_ref[...] = dinv * h


def _l0(u, hs, dinv, w0, b0):
    g = NP // _R
    return pl.pallas_call(
        _l0_body,
        grid=(g,),
        in_specs=[
            pl.BlockSpec((2, _R, 1), lambda i: (0, i, 0)),
            pl.BlockSpec((_R, 1), lambda i: (i, 0)),
            pl.BlockSpec((_R, 1), lambda i: (i, 0)),
            pl.BlockSpec((1, WF), lambda i: (0, 0)),
            pl.BlockSpec((1, WF), lambda i: (0, 0)),
        ],
        out_specs=pl.BlockSpec((_R, WF), lambda i: (i, 0)),
        out_shape=jax.ShapeDtypeStruct((NP, WF), F32),
    )(u, hs, dinv, w0, b0)


def _lmid_body(u_ref, hs_ref, dinv_ref, w_ref, b_ref, o_ref):
    dinv = dinv_ref[...]
    agg = dinv * (u_ref[0] + u_ref[1] + hs_ref[...])
    h = jnp.tanh(
        jnp.dot(agg, w_ref[...], preferred_element_type=F32) + b_ref[...]
    )
    o_ref[...] = dinv * h


def _lmid(u, hs, dinv, w, b):
    g = NP // _R
    return pl.pallas_call(
        _lmid_body,
        grid=(g,),
        in_specs=[
            pl.BlockSpec((2, _R, WF), lambda i: (0, i, 0)),
            pl.BlockSpec((_R, WF), lambda i: (i, 0)),
            pl.BlockSpec((_R, 1), lambda i: (i, 0)),
            pl.BlockSpec((WF, WF), lambda i: (0, 0)),
            pl.BlockSpec((1, WF), lambda i: (0, 0)),
        ],
        out_specs=pl.BlockSpec((_R, WF), lambda i: (i, 0)),
        out_shape=jax.ShapeDtypeStruct((NP, WF), F32),
    )(u, hs, dinv, w, b)


def _l3z_body(u_ref, hs_ref, dinv_ref, w3_ref, b3_ref, w4_ref, o_ref):
    dinv = dinv_ref[...]
    agg = dinv * (u_ref[0] + u_ref[1] + hs_ref[...])
    h4 = jnp.tanh(
        jnp.dot(agg, w3_ref[...], preferred_element_type=F32) + b3_ref[...]
    )
    z = jnp.dot(h4, w4_ref[...], preferred_element_type=F32)
    o_ref[...] = dinv * z


def _l3z(u, hs, dinv, w3, b3, w4):
    g = NP // _R
    return pl.pallas_call(
        _l3z_body,
        grid=(g,),
        in_specs=[
            pl.BlockSpec((2, _R, WF), lambda i: (0, i, 0)),
            pl.BlockSpec((_R, WF), lambda i: (i, 0)),
            pl.BlockSpec((_R, 1), lambda i: (i, 0)),
            pl.BlockSpec((WF, WF), lambda i: (0, 0)),
            pl.BlockSpec((1, WF), lambda i: (0, 0)),
            pl.BlockSpec((WF, 1), lambda i: (0, 0)),
        ],
        out_specs=pl.BlockSpec((_R, 1), lambda i: (i, 0)),
        out_shape=jax.ShapeDtypeStruct((NP, 1), F32),
    )(u, hs, dinv, w3, b3, w4)


def _fin_body(u_ref, zs_ref, dinv_ref, b4_ref, y_ref, ka_ref, kb_ref, kc_ref,
              coef_ref, k_ref, ya_ref, hsa_ref):
    dinv = dinv_ref[...]
    k = dinv * (u_ref[0] + u_ref[1] + zs_ref[...]) + b4_ref[0, 0]
    y_arg = (
        y_ref[...]
        + coef_ref[0, 0] * ka_ref[...]
        + coef_ref[0, 1] * kb_ref[...]
        + coef_ref[0, 2] * kc_ref[...]
        + coef_ref[0, 3] * k
    )
    k_ref[...] = k
    ya_ref[...] = y_arg
    hsa_ref[...] = dinv * y_arg


def _fin(u, zs, dinv, b4, y, ka, kb, kc, coef):
    return pl.pallas_call(
        _fin_body,
        out_shape=(
            jax.ShapeDtypeStruct((_NR, 128), F32),
            jax.ShapeDtypeStruct((_NR, 128), F32),
            jax.ShapeDtypeStruct((_NR, 128), F32),
        ),
    )(u, zs, dinv, b4, y, ka, kb, kc, coef)


def _padw(w, rows, cols):
    return jnp.pad(w, ((0, rows - w.shape[0]), (0, cols - w.shape[1])))


def kernel(x, edge_index, W0, b0, W1, b1, W2, b2, W3, b3, W4, b4):
    bsz, nn, _ = x.shape
    n = bsz * nn
    pad = NP - n
    y0 = x[:, :, -1].reshape(n)
    y2 = jnp.pad(y0, (0, pad)).reshape(_NR, 128)
    epad = jnp.full((EP - E,), NP - 1, jnp.int32)
    src = jnp.concatenate([edge_index[0].astype(jnp.int32), epad]).reshape(NW, NBC, K)
    dst = jnp.concatenate([edge_index[1].astype(jnp.int32), epad]).reshape(NW, NBC, K)

    w0p = _padw(W0, 1, WF)
    w1p = _padw(W1, WF, WF)
    w2p = _padw(W2, WF, WF)
    w3p = _padw(W3, WF, WF)
    w4p = _padw(W4.reshape(64, 1), WF, 1)
    b0p, b1p, b2p, b3p = (
        jnp.pad(b.reshape(1, 64), ((0, 0), (0, WF - 64))) for b in (b0, b1, b2, b3)
    )
    b4r = b4.reshape(1, 1)

    deg = _agg1(jnp.ones((NP,), F32), src, dst)
    dinv2, hs2 = _prep(deg.reshape(NC, _NR, 128), y2)
    dinvcol = dinv2.reshape(NP, 1)

    dt = 1.25
    coefs = [
        (0.0, 0.0, 0.0, dt / 3.0),
        (-dt / 3.0, 0.0, 0.0, dt),
        (dt, -dt, 0.0, dt),
        (dt / 8.0, 3.0 * dt / 8.0, 3.0 * dt / 8.0, dt / 8.0),
    ]
    coefs = [jnp.asarray(c, F32).reshape(1, 4) for c in coefs]

    preds = [y0]
    for _ in range(4):
        ks = []
        for e in range(4):
            u = _agg1(hs2.reshape(NP), src, dst).reshape(NC, NP, 1)
            h1 = _l0(u, hs2.reshape(NP, 1), dinvcol, w0p, b0p)
            u = _aggF(h1, src, dst)
            h2 = _lmid(u, h1, dinvcol, w1p, b1p)
            u = _aggF(h2, src, dst)
            h3 = _lmid(u, h2, dinvcol, w2p, b2p)
            u = _aggF(h3, src, dst)
            zs = _l3z(u, h3, dinvcol, w3p, b3p, w4p)
            u = _agg1(zs.reshape(NP), src, dst).reshape(NC, _NR, 128)
            pads = [y2, y2, y2]
            ka, kb, kc = (ks + pads)[:3]
            knew, ya2, hs2 = _fin(
                u, zs.reshape(_NR, 128), dinv2, b4r, y2, ka, kb, kc, coefs[e]
            )
            ks.append(knew)
        y2 = ya2
        preds.append(y2.reshape(NP)[:n])
    out = jnp.stack([p.reshape(nn) for p in preds], axis=-1)
    return out[None].astype(x.dtype)
